# Initial kernel scaffold; baseline (speedup 1.0000x reference)
import jax
import jax.numpy as jnp
from jax.experimental import pallas as pl


def kernel(x, edge_src, edge_dst, W_fgl, b_fgl, W_lin, b_lin):
    raise NotImplementedError("write your pallas kernel here")


# speedup vs baseline: 58.0766x; 58.0766x over previous
import functools

import jax
import jax.numpy as jnp
from jax import lax
from jax.experimental import pallas as pl
from jax.experimental.pallas import tpu as pltpu
from jax.experimental.pallas import tpu_sc as plsc

B = 128
N_IN = 784
IN_C = 3
MIDC = 16
N_OUT = 10000
E = 160000
YDIM = 10

ROWS_W = 128
N_WIN = 80
N_OUT_PAD = ROWS_W * N_WIN
NC = 2
NS = 16
WPC = N_WIN // NC
CHUNK = 2048
EPT = 10240
E_PAD = NS * EPT
E_ALLOC = E_PAD + CHUNK
JB = 1024
N_BLK = N_OUT_PAD // JB


def _sc_body(src_hbm, dst_hbm, a_hbm,
             ebuf_src, ebuf_dst, hist, all_hist, bounds, win, shared_hist):
    c = lax.axis_index("c")
    s = lax.axis_index("s")
    i16 = lax.iota(jnp.int32, 16)
    ones_i = jnp.ones((16,), jnp.int32)
    ones_f = jnp.ones((16,), jnp.float32)
    zeros_i = jnp.zeros((16,), jnp.int32)
    zeros_f = jnp.zeros((16,), jnp.float32)

    for j in range(8):
        hist[pl.ds(j * 16, 16)] = zeros_f

    def hist_round(r, carry0):
        base = pl.multiple_of(s * EPT + r * CHUNK, 8)
        pltpu.sync_copy(dst_hbm.at[pl.ds(base, CHUNK)], ebuf_dst)

        def hist_vreg(i, carry1):
            dv = ebuf_dst[pl.ds(i * 16, 16)]
            bkt = lax.shift_right_logical(dv, 7)
            plsc.addupdate_scatter(hist, [bkt], ones_f)
            return carry1

        return lax.fori_loop(0, CHUNK // 16, hist_vreg, carry0)

    lax.fori_loop(0, EPT // CHUNK, hist_round, 0)

    pltpu.sync_copy(hist, shared_hist.at[s])
    plsc.subcore_barrier()
    pltpu.sync_copy(shared_hist, all_hist)

    carry = jnp.float32(0)
    for j in range(5):
        tot = zeros_f
        for t in range(NS):
            tot = tot + all_hist[t, pl.ds(j * 16, 16)]
        cs = lax.cumsum(tot, axis=0)
        bounds[pl.ds(j * 16, 16)] = ((cs - tot) + carry).astype(jnp.int32)
        carry = carry + jnp.sum(tot)
    bounds[pl.ds(80, 16)] = jnp.broadcast_to(carry.astype(jnp.int32), (16,))

    for p in range(3):
        wl = s + 16 * p

        @pl.when(wl < WPC)
        def _process():
            w = c * WPC + wl
            bv = bounds[pl.ds(w, 16)]
            lo = bv[0]
            hi = bv[1]
            row0 = w * ROWS_W

            def zrow(i, carry1):
                for j in range(N_IN // 16):
                    win[i, pl.ds(j * 16, 16)] = zeros_f
                return carry1

            lax.fori_loop(0, ROWS_W, zrow, 0)

            lo8 = lax.bitwise_and(lo, jnp.int32(-8))
            nch = lax.shift_right_logical(hi - lo8 + (CHUNK - 1), 11)

            def chunk_body(k, carry1):
                base = pl.multiple_of(lo8 + k * CHUNK, 8)
                pltpu.sync_copy(src_hbm.at[pl.ds(base, CHUNK)], ebuf_src)
                pltpu.sync_copy(dst_hbm.at[pl.ds(base, CHUNK)], ebuf_dst)

                def vreg_body(i, carry2):
                    sv = ebuf_src[pl.ds(i * 16, 16)]
                    dv = ebuf_dst[pl.ds(i * 16, 16)]
                    e = (base + i * 16) + i16
                    m = (e >= lo) & (e < hi)
                    plsc.addupdate_scatter(win, [dv - row0, sv], ones_f,
                                           mask=m)
                    return carry2

                return lax.fori_loop(0, CHUNK // 16, vreg_body, carry1)

            lax.fori_loop(0, nch, chunk_body, 0)
            pltpu.sync_copy(win, a_hbm.at[pl.ds(row0, ROWS_W)])


def _build_a(src_pad, dst_pad):
    mesh = plsc.VectorSubcoreMesh(core_axis_name="c", subcore_axis_name="s")
    return pl.kernel(
        _sc_body,
        out_type=jax.ShapeDtypeStruct((N_OUT_PAD, N_IN), jnp.float32),
        mesh=mesh,
        compiler_params=pltpu.CompilerParams(needs_layout_passes=False),
        scratch_types=[
            pltpu.VMEM((CHUNK,), jnp.int32),
            pltpu.VMEM((CHUNK,), jnp.int32),
            pltpu.VMEM((128,), jnp.float32),
            pltpu.VMEM((NS, 128), jnp.float32),
            pltpu.VMEM((128,), jnp.int32),
            pltpu.VMEM((ROWS_W, N_IN), jnp.float32),
            pltpu.VMEM_SHARED((NS, 128), jnp.float32),
        ],
    )(src_pad, dst_pad)


def _tc_body(xt3_ref, a_ref, wf_ref, bf_ref, wl_ref, bl_ref, out_ref, xr_ref):
    g = pl.program_id(0)

    @pl.when(g == 0)
    def _init():
        for ch in range(IN_C):
            xr_ref[ch] = xt3_ref[ch].T
        out_ref[...] = jnp.broadcast_to(bl_ref[...], (B, YDIM))

    a_blk = a_ref[...]
    wf = wf_ref[...]
    bf = bf_ref[...]
    acc = jnp.zeros((JB, MIDC, B), jnp.float32)
    for ch in range(IN_C):
        m_c = jnp.dot(a_blk, xr_ref[ch], preferred_element_type=jnp.float32,
                      precision=lax.Precision.HIGHEST)
        acc = acc + m_c[:, None, :] * wf[ch][None, :, None]
    h = jnp.tanh(acc + bf[0][None, :, None])
    h2 = h.reshape(JB * MIDC, B)
    wl2 = wl_ref[...].reshape(JB * MIDC, YDIM)
    partial = lax.dot_general(h2, wl2, (((0,), (0,)), ((), ())),
                              preferred_element_type=jnp.float32,
                              precision=lax.Precision.HIGHEST)
    out_ref[...] += partial


def _dense(xt3, a, wf, bf, wl3, bl):
    return pl.pallas_call(
        _tc_body,
        grid=(N_BLK,),
        in_specs=[
            pl.BlockSpec((IN_C, B, N_IN), lambda g: (0, 0, 0)),
            pl.BlockSpec((JB, N_IN), lambda g: (g, 0)),
            pl.BlockSpec((IN_C, MIDC), lambda g: (0, 0)),
            pl.BlockSpec((1, MIDC), lambda g: (0, 0)),
            pl.BlockSpec((JB, MIDC, YDIM), lambda g: (g, 0, 0)),
            pl.BlockSpec((1, YDIM), lambda g: (0, 0)),
        ],
        out_specs=pl.BlockSpec((B, YDIM), lambda g: (0, 0)),
        out_shape=jax.ShapeDtypeStruct((B, YDIM), jnp.float32),
        scratch_shapes=[pltpu.VMEM((IN_C, N_IN, B), jnp.float32)],
    )(xt3, a, wf, bf, wl3, bl)


@jax.jit
def kernel(x, edge_src, edge_dst, W_fgl, b_fgl, W_lin, b_lin):
    src_pad = jnp.concatenate(
        [edge_src.astype(jnp.int32),
         jnp.zeros((E_ALLOC - E,), jnp.int32)])
    dst_pad = jnp.concatenate(
        [edge_dst.astype(jnp.int32),
         jnp.full((E_ALLOC - E,), N_OUT_PAD - 1, jnp.int32)])
    a = _build_a(src_pad, dst_pad)

    xt3 = jnp.transpose(x, (2, 0, 1))
    wl3 = jnp.pad(W_lin.reshape(N_OUT, MIDC, YDIM),
                  ((0, N_OUT_PAD - N_OUT), (0, 0), (0, 0)))
    return _dense(xt3, a, W_fgl, b_fgl.reshape(1, MIDC), wl3,
                  b_lin.reshape(1, YDIM))


# speedup vs baseline: 110.7738x; 1.9074x over previous
import functools

import jax
import jax.numpy as jnp
from jax import lax
from jax.experimental import pallas as pl
from jax.experimental.pallas import tpu as pltpu
from jax.experimental.pallas import tpu_sc as plsc

B = 128
N_IN = 784
IN_C = 3
MIDC = 16
N_OUT = 10000
E = 160000
YDIM = 10

ROWS_W = 128
N_WIN = 80
N_OUT_PAD = ROWS_W * N_WIN
NC = 2
NS = 16
WPC = N_WIN // NC
CHUNK = 2048
EPT = 10240
E_PAD = NS * EPT
E_ALLOC = E_PAD + CHUNK
JB = 1024
N_BLK = N_OUT_PAD // JB


def _sc_body(src_hbm, dst_hbm, a_hbm,
             ebuf_src, ebuf_dst, hist, all_hist, bounds, win, shared_hist):
    c = lax.axis_index("c")
    s = lax.axis_index("s")
    i16 = lax.iota(jnp.int32, 16)
    ones_i = jnp.ones((16,), jnp.int32)
    ones_f = jnp.ones((16,), jnp.float32)
    zeros_i = jnp.zeros((16,), jnp.int32)
    zeros_f = jnp.zeros((16,), jnp.float32)

    for j in range(8):
        hist[pl.ds(j * 16, 16)] = zeros_f

    def hist_round(r, carry0):
        base = pl.multiple_of(s * EPT + r * CHUNK, 8)
        pltpu.sync_copy(dst_hbm.at[pl.ds(base, CHUNK)], ebuf_dst)

        def hist_vreg(i, carry1):
            dv = ebuf_dst[pl.ds(i * 16, 16)]
            bkt = lax.shift_right_logical(dv, 7)
            plsc.addupdate_scatter(hist, [bkt], ones_f)
            return carry1

        return lax.fori_loop(0, CHUNK // 16, hist_vreg, carry0)

    lax.fori_loop(0, EPT // CHUNK, hist_round, 0)

    pltpu.sync_copy(hist, shared_hist.at[s])
    plsc.subcore_barrier()
    pltpu.sync_copy(shared_hist, all_hist)

    carry = jnp.float32(0)
    for j in range(5):
        tot = zeros_f
        for t in range(NS):
            tot = tot + all_hist[t, pl.ds(j * 16, 16)]
        cs = lax.cumsum(tot, axis=0)
        bounds[pl.ds(j * 16, 16)] = ((cs - tot) + carry).astype(jnp.int32)
        carry = carry + jnp.sum(tot)
    bounds[pl.ds(80, 16)] = jnp.broadcast_to(carry.astype(jnp.int32), (16,))

    for p in range(3):
        wl = s + 16 * p

        @pl.when(wl < WPC)
        def _process():
            w = c * WPC + wl
            bv = bounds[pl.ds(w, 16)]
            lo = bv[0]
            hi = bv[1]
            row0 = w * ROWS_W

            def zrow(i, carry1):
                for j in range(N_IN // 16):
                    win[i, pl.ds(j * 16, 16)] = zeros_f
                return carry1

            lax.fori_loop(0, ROWS_W, zrow, 0)

            lo8 = lax.bitwise_and(lo, jnp.int32(-8))
            nch = lax.shift_right_logical(hi - lo8 + (CHUNK - 1), 11)

            def chunk_body(k, carry1):
                base = pl.multiple_of(lo8 + k * CHUNK, 8)
                pltpu.sync_copy(src_hbm.at[pl.ds(base, CHUNK)], ebuf_src)
                pltpu.sync_copy(dst_hbm.at[pl.ds(base, CHUNK)], ebuf_dst)

                def vreg_body(i, carry2):
                    sv = ebuf_src[pl.ds(i * 16, 16)]
                    dv = ebuf_dst[pl.ds(i * 16, 16)]
                    e = (base + i * 16) + i16
                    m = (e >= lo) & (e < hi)
                    plsc.addupdate_scatter(win, [dv - row0, sv], ones_f,
                                           mask=m)
                    return carry2

                return lax.fori_loop(0, CHUNK // 16, vreg_body, carry1)

            lax.fori_loop(0, nch, chunk_body, 0)
            pltpu.sync_copy(win, a_hbm.at[pl.ds(row0, ROWS_W)])


def _build_a(src_pad, dst_pad):
    mesh = plsc.VectorSubcoreMesh(core_axis_name="c", subcore_axis_name="s")
    return pl.kernel(
        _sc_body,
        out_type=jax.ShapeDtypeStruct((N_OUT_PAD, N_IN), jnp.float32),
        mesh=mesh,
        compiler_params=pltpu.CompilerParams(needs_layout_passes=False),
        scratch_types=[
            pltpu.VMEM((CHUNK,), jnp.int32),
            pltpu.VMEM((CHUNK,), jnp.int32),
            pltpu.VMEM((128,), jnp.float32),
            pltpu.VMEM((NS, 128), jnp.float32),
            pltpu.VMEM((128,), jnp.int32),
            pltpu.VMEM((ROWS_W, N_IN), jnp.float32),
            pltpu.VMEM_SHARED((NS, 128), jnp.float32),
        ],
    )(src_pad, dst_pad)


def _tc_body(xt3_ref, a_ref, wf_ref, bf_ref, wl_ref, bl_ref, out_ref, xr_ref):
    g = pl.program_id(0)

    @pl.when(g == 0)
    def _init():
        for ch in range(IN_C):
            xr_ref[ch] = xt3_ref[ch].T
        out_ref[...] = jnp.broadcast_to(bl_ref[...], (B, YDIM))

    a_blk = a_ref[...]
    wf = wf_ref[...]
    bf = bf_ref[...]
    acc = jnp.zeros((JB, MIDC, B), jnp.float32)
    for ch in range(IN_C):
        m_c = jnp.dot(a_blk, xr_ref[ch], preferred_element_type=jnp.float32)
        acc = acc + m_c[:, None, :] * wf[ch][None, :, None]
    h = jnp.tanh(acc + bf[0][None, :, None])
    h2 = h.reshape(JB * MIDC, B)
    wl2 = wl_ref[...].reshape(JB * MIDC, YDIM)
    partial = lax.dot_general(h2, wl2, (((0,), (0,)), ((), ())),
                              preferred_element_type=jnp.float32)
    out_ref[...] += partial


def _dense(xt3, a, wf, bf, wl3, bl):
    return pl.pallas_call(
        _tc_body,
        grid=(N_BLK,),
        in_specs=[
            pl.BlockSpec((IN_C, B, N_IN), lambda g: (0, 0, 0)),
            pl.BlockSpec((JB, N_IN), lambda g: (g, 0)),
            pl.BlockSpec((IN_C, MIDC), lambda g: (0, 0)),
            pl.BlockSpec((1, MIDC), lambda g: (0, 0)),
            pl.BlockSpec((JB, MIDC, YDIM), lambda g: (g, 0, 0)),
            pl.BlockSpec((1, YDIM), lambda g: (0, 0)),
        ],
        out_specs=pl.BlockSpec((B, YDIM), lambda g: (0, 0)),
        out_shape=jax.ShapeDtypeStruct((B, YDIM), jnp.float32),
        scratch_shapes=[pltpu.VMEM((IN_C, N_IN, B), jnp.float32)],
    )(xt3, a, wf, bf, wl3, bl)


@jax.jit
def kernel(x, edge_src, edge_dst, W_fgl, b_fgl, W_lin, b_lin):
    src_pad = jnp.concatenate(
        [edge_src.astype(jnp.int32),
         jnp.zeros((E_ALLOC - E,), jnp.int32)])
    dst_pad = jnp.concatenate(
        [edge_dst.astype(jnp.int32),
         jnp.full((E_ALLOC - E,), N_OUT_PAD - 1, jnp.int32)])
    a = _build_a(src_pad, dst_pad)

    xt3 = jnp.transpose(x, (2, 0, 1))
    wl3 = jnp.pad(W_lin.reshape(N_OUT, MIDC, YDIM),
                  ((0, N_OUT_PAD - N_OUT), (0, 0), (0, 0)))
    return _dense(xt3, a, W_fgl, b_fgl.reshape(1, MIDC), wl3,
                  b_lin.reshape(1, YDIM))


# speedup vs baseline: 111.3877x; 1.0055x over previous
import functools

import jax
import jax.numpy as jnp
from jax import lax
from jax.experimental import pallas as pl
from jax.experimental.pallas import tpu as pltpu
from jax.experimental.pallas import tpu_sc as plsc

B = 128
N_IN = 784
IN_C = 3
MIDC = 16
N_OUT = 10000
E = 160000
YDIM = 10

ROWS_W = 128
N_WIN = 80
N_OUT_PAD = ROWS_W * N_WIN
NC = 2
NS = 16
WPC = N_WIN // NC
CHUNK = 2048
EPT = 10240
E_PAD = NS * EPT
E_ALLOC = E_PAD + CHUNK
JB = 1024
N_BLK = N_OUT_PAD // JB


def _sc_body(src_hbm, dst_hbm, a_hbm,
             ebuf_src, ebuf_dst, hist, all_hist, bounds, win, shared_hist):
    c = lax.axis_index("c")
    s = lax.axis_index("s")
    i16 = lax.iota(jnp.int32, 16)
    ones_f = jnp.ones((16,), jnp.float32)
    zeros_f = jnp.zeros((16,), jnp.float32)

    for j in range(8):
        hist[pl.ds(j * 16, 16)] = zeros_f

    def hist_round(r, carry0):
        base = pl.multiple_of(s * EPT + r * CHUNK, 8)
        pltpu.sync_copy(dst_hbm.at[pl.ds(base, CHUNK)], ebuf_dst)

        def hist_vreg(i, carry1):
            dv = ebuf_dst[pl.ds(i * 16, 16)]
            bkt = lax.shift_right_logical(dv, 7)
            plsc.addupdate_scatter(hist, [bkt], ones_f)
            return carry1

        return lax.fori_loop(0, CHUNK // 16, hist_vreg, carry0)

    lax.fori_loop(0, EPT // CHUNK, hist_round, 0)

    pltpu.sync_copy(hist, shared_hist.at[s])
    plsc.subcore_barrier()
    pltpu.sync_copy(shared_hist, all_hist)

    carry = jnp.float32(0)
    for j in range(N_WIN // 16):
        tot = zeros_f
        for t in range(NS):
            tot = tot + all_hist[t, pl.ds(j * 16, 16)]
        cs = lax.cumsum(tot, axis=0)
        bounds[pl.ds(j * 16, 16)] = ((cs - tot) + carry).astype(jnp.int32)
        carry = carry + jnp.sum(tot)
    bounds[pl.ds(N_WIN, 16)] = jnp.broadcast_to(carry.astype(jnp.int32), (16,))

    for p in range(3):
        wl = s + 16 * p

        @pl.when(wl < WPC)
        def _process():
            w = c * WPC + wl
            bv = bounds[pl.ds(w, 16)]
            lo = bv[0]
            hi = bv[1]
            row0 = w * ROWS_W

            def zrow(i, carry1):
                for j in range(N_IN // 16):
                    win[i, pl.ds(j * 16, 16)] = zeros_f
                return carry1

            lax.fori_loop(0, ROWS_W, zrow, 0)

            lo8 = lax.bitwise_and(lo, jnp.int32(-8))
            nch = lax.shift_right_logical(hi - lo8 + (CHUNK - 1), 11)

            def chunk_body(k, carry1):
                base = pl.multiple_of(lo8 + k * CHUNK, 8)
                pltpu.sync_copy(src_hbm.at[pl.ds(base, CHUNK)], ebuf_src)
                pltpu.sync_copy(dst_hbm.at[pl.ds(base, CHUNK)], ebuf_dst)
                nv = lax.min(jnp.int32(CHUNK // 16),
                             lax.shift_right_logical(hi - base + 15, 4))

                def vreg_body(i, carry2):
                    sv = ebuf_src[pl.ds(i * 16, 16)]
                    dv = ebuf_dst[pl.ds(i * 16, 16)]
                    e = (base + i * 16) + i16
                    m = (e >= lo) & (e < hi)
                    plsc.addupdate_scatter(win, [dv - row0, sv], ones_f,
                                           mask=m)
                    return carry2

                return lax.fori_loop(0, nv, vreg_body, carry1)

            lax.fori_loop(0, nch, chunk_body, 0)
            pltpu.sync_copy(win, a_hbm.at[pl.ds(row0, ROWS_W)])


def _build_a(src_pad, dst_pad):
    mesh = plsc.VectorSubcoreMesh(core_axis_name="c", subcore_axis_name="s")
    return pl.kernel(
        _sc_body,
        out_type=jax.ShapeDtypeStruct((N_OUT_PAD, N_IN), jnp.float32),
        mesh=mesh,
        compiler_params=pltpu.CompilerParams(needs_layout_passes=False),
        scratch_types=[
            pltpu.VMEM((CHUNK,), jnp.int32),
            pltpu.VMEM((CHUNK,), jnp.int32),
            pltpu.VMEM((128,), jnp.float32),
            pltpu.VMEM((NS, 128), jnp.float32),
            pltpu.VMEM((128,), jnp.int32),
            pltpu.VMEM((ROWS_W, N_IN), jnp.float32),
            pltpu.VMEM_SHARED((NS, 128), jnp.float32),
        ],
    )(src_pad, dst_pad)


def _tc_body(xt3_ref, a_ref, wf_ref, bf_ref, wl_ref, bl_ref, out_ref, xr_ref):
    g = pl.program_id(0)

    @pl.when(g == 0)
    def _init():
        for ch in range(IN_C):
            xr_ref[ch] = xt3_ref[ch].T
        out_ref[...] = jnp.broadcast_to(bl_ref[...], (B, YDIM))

    a_blk = a_ref[...]
    wf = wf_ref[...]
    bf = bf_ref[...]
    acc = jnp.zeros((JB, MIDC, B), jnp.float32)
    for ch in range(IN_C):
        m_c = jnp.dot(a_blk, xr_ref[ch], preferred_element_type=jnp.float32)
        acc = acc + m_c[:, None, :] * wf[ch][None, :, None]
    h = jnp.tanh(acc + bf[0][None, :, None])
    h2 = h.reshape(JB * MIDC, B)
    wl2 = wl_ref[...].reshape(JB * MIDC, YDIM)
    partial = lax.dot_general(h2, wl2, (((0,), (0,)), ((), ())),
                              preferred_element_type=jnp.float32)
    out_ref[...] += partial


def _dense(xt3, a, wf, bf, wl3, bl):
    return pl.pallas_call(
        _tc_body,
        grid=(N_BLK,),
        in_specs=[
            pl.BlockSpec((IN_C, B, N_IN), lambda g: (0, 0, 0)),
            pl.BlockSpec((JB, N_IN), lambda g: (g, 0)),
            pl.BlockSpec((IN_C, MIDC), lambda g: (0, 0)),
            pl.BlockSpec((1, MIDC), lambda g: (0, 0)),
            pl.BlockSpec((JB, MIDC, YDIM), lambda g: (g, 0, 0)),
            pl.BlockSpec((1, YDIM), lambda g: (0, 0)),
        ],
        out_specs=pl.BlockSpec((B, YDIM), lambda g: (0, 0)),
        out_shape=jax.ShapeDtypeStruct((B, YDIM), jnp.float32),
        scratch_shapes=[pltpu.VMEM((IN_C, N_IN, B), jnp.float32)],
    )(xt3, a, wf, bf, wl3, bl)


@jax.jit
def kernel(x, edge_src, edge_dst, W_fgl, b_fgl, W_lin, b_lin):
    src_pad = jnp.concatenate(
        [edge_src.astype(jnp.int32),
         jnp.zeros((E_ALLOC - E,), jnp.int32)])
    dst_pad = jnp.concatenate(
        [edge_dst.astype(jnp.int32),
         jnp.full((E_ALLOC - E,), N_OUT_PAD - 1, jnp.int32)])
    a = _build_a(src_pad, dst_pad)

    xt3 = jnp.transpose(x, (2, 0, 1))
    wl3 = jnp.pad(W_lin.reshape(N_OUT, MIDC, YDIM),
                  ((0, N_OUT_PAD - N_OUT), (0, 0), (0, 0)))
    return _dense(xt3, a, W_fgl, b_fgl.reshape(1, MIDC), wl3,
                  b_lin.reshape(1, YDIM))


# speedup vs baseline: 112.0077x; 1.0056x over previous
import functools

import jax
import jax.numpy as jnp
from jax import lax
from jax.experimental import pallas as pl
from jax.experimental.pallas import tpu as pltpu
from jax.experimental.pallas import tpu_sc as plsc

B = 128
N_IN = 784
IN_C = 3
MIDC = 16
N_OUT = 10000
E = 160000
YDIM = 10

ROWS_W = 128
N_WIN = 80
N_OUT_PAD = ROWS_W * N_WIN
NC = 2
NS = 16
WPC = N_WIN // NC
HALF_R = 64
CHUNK = 2048
EPT = 10240
E_PAD = NS * EPT
E_ALLOC = E_PAD + CHUNK
JB = 1024
N_BLK = N_OUT_PAD // JB


def _sc_body(src_hbm, dst_hbm, a_hbm,
             ebuf_src, ebuf_dst, hist, all_hist, bounds, win_a, win_b, sem,
             shared_hist):
    c = lax.axis_index("c")
    s = lax.axis_index("s")
    i16 = lax.iota(jnp.int32, 16)
    ones_f = jnp.ones((16,), jnp.float32)
    zeros_f = jnp.zeros((16,), jnp.float32)

    for j in range(8):
        hist[pl.ds(j * 16, 16)] = zeros_f

    def hist_round(r, carry0):
        base = pl.multiple_of(s * EPT + r * CHUNK, 8)
        pltpu.sync_copy(dst_hbm.at[pl.ds(base, CHUNK)], ebuf_dst)

        def hist_vreg(i, carry1):
            dv = ebuf_dst[pl.ds(i * 16, 16)]
            bkt = lax.shift_right_logical(dv, 7)
            plsc.addupdate_scatter(hist, [bkt], ones_f)
            return carry1

        return lax.fori_loop(0, CHUNK // 16, hist_vreg, carry0)

    lax.fori_loop(0, EPT // CHUNK, hist_round, 0)

    pltpu.sync_copy(hist, shared_hist.at[s])
    plsc.subcore_barrier()
    pltpu.sync_copy(shared_hist, all_hist)

    carry = jnp.float32(0)
    for j in range(N_WIN // 16):
        tot = zeros_f
        for t in range(NS):
            tot = tot + all_hist[t, pl.ds(j * 16, 16)]
        cs = lax.cumsum(tot, axis=0)
        bounds[pl.ds(j * 16, 16)] = ((cs - tot) + carry).astype(jnp.int32)
        carry = carry + jnp.sum(tot)
    bounds[pl.ds(N_WIN, 16)] = jnp.broadcast_to(carry.astype(jnp.int32), (16,))

    def zero_win(wref):
        def zrow(i, carry1):
            for j in range(N_IN // 16):
                wref[i, pl.ds(j * 16, 16)] = zeros_f
            return carry1
        lax.fori_loop(0, HALF_R, zrow, 0)

    def scatter_half(wref, ghalf):
        w = lax.shift_right_logical(ghalf, 1)
        bv = bounds[pl.ds(w, 16)]
        lo = bv[0]
        hi = bv[1]
        row0 = ghalf * HALF_R
        lo8 = lax.bitwise_and(lo, jnp.int32(-8))
        nch = lax.shift_right_logical(hi - lo8 + (CHUNK - 1), 11)

        def chunk_body(k, carry1):
            base = pl.multiple_of(lo8 + k * CHUNK, 8)
            pltpu.sync_copy(src_hbm.at[pl.ds(base, CHUNK)], ebuf_src)
            pltpu.sync_copy(dst_hbm.at[pl.ds(base, CHUNK)], ebuf_dst)
            nv = lax.min(jnp.int32(CHUNK // 16),
                         lax.shift_right_logical(hi - base + 15, 4))

            def vreg_body(i, carry2):
                sv = ebuf_src[pl.ds(i * 16, 16)]
                dv = ebuf_dst[pl.ds(i * 16, 16)]
                e = (base + i * 16) + i16
                r = dv - row0
                m = (e >= lo) & (e < hi) & (r >= 0) & (r < HALF_R)
                plsc.addupdate_scatter(wref, [r, sv], ones_f, mask=m)
                return carry2

            return lax.fori_loop(0, nv, vreg_body, carry1)

        lax.fori_loop(0, nch, chunk_body, 0)
        return row0

    bufs = (win_a, win_b)
    zero_win(win_a)
    copies = [None, None]
    for q in range(5):
        buf = bufs[q % 2]
        ghalf = c * (N_WIN * 2 // NC) + s + NS * q
        row0 = scatter_half(buf, ghalf)
        copies[q % 2] = pltpu.async_copy(
            buf, a_hbm.at[pl.ds(row0, HALF_R)], sem)
        if q < 4:
            if copies[(q + 1) % 2] is not None:
                copies[(q + 1) % 2].wait()
            zero_win(bufs[(q + 1) % 2])
    copies[1].wait()
    copies[0].wait()


def _build_a(src_pad, dst_pad):
    mesh = plsc.VectorSubcoreMesh(core_axis_name="c", subcore_axis_name="s")
    return pl.kernel(
        _sc_body,
        out_type=jax.ShapeDtypeStruct((N_OUT_PAD, N_IN), jnp.float32),
        mesh=mesh,
        compiler_params=pltpu.CompilerParams(needs_layout_passes=False),
        scratch_types=[
            pltpu.VMEM((CHUNK,), jnp.int32),
            pltpu.VMEM((CHUNK,), jnp.int32),
            pltpu.VMEM((128,), jnp.float32),
            pltpu.VMEM((NS, 128), jnp.float32),
            pltpu.VMEM((128,), jnp.int32),
            pltpu.VMEM((HALF_R, N_IN), jnp.float32),
            pltpu.VMEM((HALF_R, N_IN), jnp.float32),
            pltpu.SemaphoreType.DMA,
            pltpu.VMEM_SHARED((NS, 128), jnp.float32),
        ],
    )(src_pad, dst_pad)


def _tc_body(xt3_ref, a_ref, wf_ref, bf_ref, wl_ref, bl_ref, out_ref, xr_ref):
    g = pl.program_id(0)

    @pl.when(g == 0)
    def _init():
        for ch in range(IN_C):
            xr_ref[ch] = xt3_ref[ch].T
        out_ref[...] = jnp.broadcast_to(bl_ref[...], (B, YDIM))

    a_blk = a_ref[...]
    wf = wf_ref[...]
    bf = bf_ref[...]
    acc = jnp.zeros((JB, MIDC, B), jnp.float32)
    for ch in range(IN_C):
        m_c = jnp.dot(a_blk, xr_ref[ch], preferred_element_type=jnp.float32)
        acc = acc + m_c[:, None, :] * wf[ch][None, :, None]
    h = jnp.tanh(acc + bf[0][None, :, None])
    h2 = h.reshape(JB * MIDC, B)
    wl2 = wl_ref[...].reshape(JB * MIDC, YDIM)
    partial = lax.dot_general(h2, wl2, (((0,), (0,)), ((), ())),
                              preferred_element_type=jnp.float32)
    out_ref[...] += partial


def _dense(xt3, a, wf, bf, wl3, bl):
    return pl.pallas_call(
        _tc_body,
        grid=(N_BLK,),
        in_specs=[
            pl.BlockSpec((IN_C, B, N_IN), lambda g: (0, 0, 0)),
            pl.BlockSpec((JB, N_IN), lambda g: (g, 0)),
            pl.BlockSpec((IN_C, MIDC), lambda g: (0, 0)),
            pl.BlockSpec((1, MIDC), lambda g: (0, 0)),
            pl.BlockSpec((JB, MIDC, YDIM), lambda g: (g, 0, 0)),
            pl.BlockSpec((1, YDIM), lambda g: (0, 0)),
        ],
        out_specs=pl.BlockSpec((B, YDIM), lambda g: (0, 0)),
        out_shape=jax.ShapeDtypeStruct((B, YDIM), jnp.float32),
        scratch_shapes=[pltpu.VMEM((IN_C, N_IN, B), jnp.float32)],
    )(xt3, a, wf, bf, wl3, bl)


@jax.jit
def kernel(x, edge_src, edge_dst, W_fgl, b_fgl, W_lin, b_lin):
    src_pad = jnp.concatenate(
        [edge_src.astype(jnp.int32),
         jnp.zeros((E_ALLOC - E,), jnp.int32)])
    dst_pad = jnp.concatenate(
        [edge_dst.astype(jnp.int32),
         jnp.full((E_ALLOC - E,), N_OUT_PAD - 1, jnp.int32)])
    a = _build_a(src_pad, dst_pad)

    xt3 = jnp.transpose(x, (2, 0, 1))
    wl3 = jnp.pad(W_lin.reshape(N_OUT, MIDC, YDIM),
                  ((0, N_OUT_PAD - N_OUT), (0, 0), (0, 0)))
    return _dense(xt3, a, W_fgl, b_fgl.reshape(1, MIDC), wl3,
                  b_lin.reshape(1, YDIM))


# speedup vs baseline: 114.2356x; 1.0199x over previous
import functools

import jax
import jax.numpy as jnp
from jax import lax
from jax.experimental import pallas as pl
from jax.experimental.pallas import tpu as pltpu
from jax.experimental.pallas import tpu_sc as plsc

B = 128
N_IN = 784
IN_C = 3
MIDC = 16
N_OUT = 10000
E = 160000
YDIM = 10

ROWS_W = 128
N_WIN = 80
N_OUT_PAD = ROWS_W * N_WIN
NC = 2
NS = 16
WPC = N_WIN // NC
HALF_R = 64
CHUNK = 2048
EPT = 10240
E_PAD = NS * EPT
E_ALLOC = E_PAD + CHUNK
JB = 1024
N_BLK = N_OUT_PAD // JB


def _sc_body(src_hbm, dst_hbm, a_hbm,
             ebuf_src, ebuf_dst, hist, all_hist, bounds, win_a, win_b, sem,
             shared_hist):
    c = lax.axis_index("c")
    s = lax.axis_index("s")
    i16 = lax.iota(jnp.int32, 16)
    ones_f = jnp.ones((16,), jnp.float32)
    zeros_f = jnp.zeros((16,), jnp.float32)

    for j in range(8):
        hist[pl.ds(j * 16, 16)] = zeros_f

    def hist_round(r, carry0):
        base = pl.multiple_of(s * EPT + r * CHUNK, 8)
        pltpu.sync_copy(dst_hbm.at[pl.ds(base, CHUNK)], ebuf_dst)

        def hist_vreg(i, carry1):
            dv = ebuf_dst[pl.ds(i * 16, 16)]
            bkt = lax.shift_right_logical(dv, 7)
            plsc.addupdate_scatter(hist, [bkt], ones_f)
            return carry1

        return lax.fori_loop(0, CHUNK // 16, hist_vreg, carry0)

    lax.fori_loop(0, EPT // CHUNK, hist_round, 0)

    pltpu.sync_copy(hist, shared_hist.at[s])
    plsc.subcore_barrier()
    pltpu.sync_copy(shared_hist, all_hist)

    carry = jnp.float32(0)
    for j in range(N_WIN // 16):
        tot = zeros_f
        for t in range(NS):
            tot = tot + all_hist[t, pl.ds(j * 16, 16)]
        cs = lax.cumsum(tot, axis=0)
        bounds[pl.ds(j * 16, 16)] = ((cs - tot) + carry).astype(jnp.int32)
        carry = carry + jnp.sum(tot)
    bounds[pl.ds(N_WIN, 16)] = jnp.broadcast_to(carry.astype(jnp.int32), (16,))

    def zero_win(wref):
        def zrow(i, carry1):
            for j in range(N_IN // 16):
                wref[i, pl.ds(j * 16, 16)] = zeros_f
            return carry1
        lax.fori_loop(0, HALF_R, zrow, 0)

    def scatter_half(wref, ghalf):
        w = lax.shift_right_logical(ghalf, 1)
        bv = bounds[pl.ds(w, 16)]
        lo = bv[0]
        hi = bv[1]
        row0 = ghalf * HALF_R
        lo8 = lax.bitwise_and(lo, jnp.int32(-8))
        nch = lax.shift_right_logical(hi - lo8 + (CHUNK - 1), 11)

        def chunk_body(k, carry1):
            base = pl.multiple_of(lo8 + k * CHUNK, 8)
            pltpu.sync_copy(src_hbm.at[pl.ds(base, CHUNK)], ebuf_src)
            pltpu.sync_copy(dst_hbm.at[pl.ds(base, CHUNK)], ebuf_dst)
            nv = lax.min(jnp.int32(CHUNK // 16),
                         lax.shift_right_logical(hi - base + 15, 4))

            def vreg_body(i, carry2):
                sv = ebuf_src[pl.ds(i * 16, 16)]
                dv = ebuf_dst[pl.ds(i * 16, 16)]
                e = (base + i * 16) + i16
                r = dv - row0
                m = (e >= lo) & (e < hi) & (r >= 0) & (r < HALF_R)
                plsc.addupdate_scatter(wref, [r, sv], ones_f, mask=m)
                return carry2

            return lax.fori_loop(0, nv, vreg_body, carry1)

        lax.fori_loop(0, nch, chunk_body, 0)
        return row0

    bufs = (win_a, win_b)
    zero_win(win_a)
    copies = [None, None]
    for q in range(5):
        buf = bufs[q % 2]
        ghalf = c * (N_WIN * 2 // NC) + s + NS * q
        row0 = scatter_half(buf, ghalf)
        copies[q % 2] = pltpu.async_copy(
            buf, a_hbm.at[pl.ds(row0, HALF_R)], sem)
        if q < 4:
            if copies[(q + 1) % 2] is not None:
                copies[(q + 1) % 2].wait()
            zero_win(bufs[(q + 1) % 2])
    copies[1].wait()
    copies[0].wait()


def _build_a(src_pad, dst_pad):
    mesh = plsc.VectorSubcoreMesh(core_axis_name="c", subcore_axis_name="s")
    return pl.kernel(
        _sc_body,
        out_type=jax.ShapeDtypeStruct((N_OUT_PAD, N_IN), jnp.float32),
        mesh=mesh,
        compiler_params=pltpu.CompilerParams(needs_layout_passes=False),
        scratch_types=[
            pltpu.VMEM((CHUNK,), jnp.int32),
            pltpu.VMEM((CHUNK,), jnp.int32),
            pltpu.VMEM((128,), jnp.float32),
            pltpu.VMEM((NS, 128), jnp.float32),
            pltpu.VMEM((128,), jnp.int32),
            pltpu.VMEM((HALF_R, N_IN), jnp.float32),
            pltpu.VMEM((HALF_R, N_IN), jnp.float32),
            pltpu.SemaphoreType.DMA,
            pltpu.VMEM_SHARED((NS, 128), jnp.float32),
        ],
    )(src_pad, dst_pad)


def _tc_body(xt3_ref, a_ref, wf_ref, bf_ref, wl_ref, bl_ref, out_ref, xr_ref):
    g = pl.program_id(0)

    @pl.when(g == 0)
    def _init():
        for ch in range(IN_C):
            xr_ref[:, ch * B:(ch + 1) * B] = xt3_ref[ch].T
        out_ref[...] = jnp.broadcast_to(bl_ref[...], (B, YDIM))

    a_blk = a_ref[...]
    wf = wf_ref[...]
    bf = bf_ref[...]
    m = jnp.dot(a_blk, xr_ref[...], preferred_element_type=jnp.float32)
    acc = jnp.zeros((JB, MIDC, B), jnp.float32)
    for ch in range(IN_C):
        m_c = m[:, ch * B:(ch + 1) * B]
        acc = acc + m_c[:, None, :] * wf[ch][None, :, None]
    h = jnp.tanh(acc + bf[0][None, :, None])
    h2 = h.reshape(JB * MIDC, B)
    wl2 = wl_ref[...].reshape(JB * MIDC, YDIM)
    partial = lax.dot_general(h2, wl2, (((0,), (0,)), ((), ())),
                              preferred_element_type=jnp.float32)
    out_ref[...] += partial


def _dense(xt3, a, wf, bf, wl3, bl):
    return pl.pallas_call(
        _tc_body,
        grid=(N_BLK,),
        in_specs=[
            pl.BlockSpec((IN_C, B, N_IN), lambda g: (0, 0, 0)),
            pl.BlockSpec((JB, N_IN), lambda g: (g, 0)),
            pl.BlockSpec((IN_C, MIDC), lambda g: (0, 0)),
            pl.BlockSpec((1, MIDC), lambda g: (0, 0)),
            pl.BlockSpec((JB, MIDC, YDIM), lambda g: (g, 0, 0)),
            pl.BlockSpec((1, YDIM), lambda g: (0, 0)),
        ],
        out_specs=pl.BlockSpec((B, YDIM), lambda g: (0, 0)),
        out_shape=jax.ShapeDtypeStruct((B, YDIM), jnp.float32),
        scratch_shapes=[pltpu.VMEM((N_IN, IN_C * B), jnp.float32)],
    )(xt3, a, wf, bf, wl3, bl)


@jax.jit
def kernel(x, edge_src, edge_dst, W_fgl, b_fgl, W_lin, b_lin):
    src_pad = jnp.concatenate(
        [edge_src.astype(jnp.int32),
         jnp.zeros((E_ALLOC - E,), jnp.int32)])
    dst_pad = jnp.concatenate(
        [edge_dst.astype(jnp.int32),
         jnp.full((E_ALLOC - E,), N_OUT_PAD - 1, jnp.int32)])
    a = _build_a(src_pad, dst_pad)

    xt3 = jnp.transpose(x, (2, 0, 1))
    wl3 = jnp.pad(W_lin.reshape(N_OUT, MIDC, YDIM),
                  ((0, N_OUT_PAD - N_OUT), (0, 0), (0, 0)))
    return _dense(xt3, a, W_fgl, b_fgl.reshape(1, MIDC), wl3,
                  b_lin.reshape(1, YDIM))


# speedup vs baseline: 117.0352x; 1.0245x over previous
import functools

import jax
import jax.numpy as jnp
from jax import lax
from jax.experimental import pallas as pl
from jax.experimental.pallas import tpu as pltpu
from jax.experimental.pallas import tpu_sc as plsc

B = 128
N_IN = 784
IN_C = 3
MIDC = 16
N_OUT = 10000
E = 160000
YDIM = 10

ROWS_W = 128
N_WIN = 80
N_OUT_PAD = ROWS_W * N_WIN
NC = 2
NS = 16
WPC = N_WIN // NC
HALF_R = 64
CHUNK = 2048
EPT = 10240
E_PAD = NS * EPT
E_ALLOC = E_PAD + CHUNK
JB = 1024
N_BLK = N_OUT_PAD // JB


def _sc_body(src_hbm, dst_hbm, a_hbm,
             ebuf_src, ebuf_dst, hist, all_hist, bounds, win_a, win_b, sem,
             shared_hist):
    c = lax.axis_index("c")
    s = lax.axis_index("s")
    i16 = lax.iota(jnp.int32, 16)
    ones_f = jnp.ones((16,), jnp.float32)
    zeros_f = jnp.zeros((16,), jnp.float32)

    for j in range(8):
        hist[pl.ds(j * 16, 16)] = zeros_f

    hbufs = (ebuf_dst, ebuf_src)
    hcopies = [None, None]
    hcopies[0] = pltpu.async_copy(
        dst_hbm.at[pl.ds(pl.multiple_of(s * EPT, 8), CHUNK)], hbufs[0], sem)
    for r in range(EPT // CHUNK):
        cur = hbufs[r % 2]
        if r + 1 < EPT // CHUNK:
            base = pl.multiple_of(s * EPT + (r + 1) * CHUNK, 8)
            hcopies[(r + 1) % 2] = pltpu.async_copy(
                dst_hbm.at[pl.ds(base, CHUNK)], hbufs[(r + 1) % 2], sem)
        hcopies[r % 2].wait()

        def hist_vreg(i, carry1, cur=cur):
            for u in range(4):
                dv = cur[pl.ds(i * 64 + u * 16, 16)]
                bkt = lax.shift_right_logical(dv, 7)
                plsc.addupdate_scatter(hist, [bkt], ones_f)
            return carry1

        lax.fori_loop(0, CHUNK // 64, hist_vreg, 0)

    pltpu.sync_copy(hist, shared_hist.at[s])
    plsc.subcore_barrier()
    pltpu.sync_copy(shared_hist, all_hist)

    carry = jnp.float32(0)
    for j in range(N_WIN // 16):
        tot = zeros_f
        for t in range(NS):
            tot = tot + all_hist[t, pl.ds(j * 16, 16)]
        cs = lax.cumsum(tot, axis=0)
        bounds[pl.ds(j * 16, 16)] = ((cs - tot) + carry).astype(jnp.int32)
        carry = carry + jnp.sum(tot)
    bounds[pl.ds(N_WIN, 16)] = jnp.broadcast_to(carry.astype(jnp.int32), (16,))

    def zero_win(wref):
        def zrow(i, carry1):
            for j in range(N_IN // 16):
                wref[i, pl.ds(j * 16, 16)] = zeros_f
            return carry1
        lax.fori_loop(0, HALF_R, zrow, 0)

    def scatter_half(wref, ghalf):
        w = lax.shift_right_logical(ghalf, 1)
        bv = bounds[pl.ds(w, 16)]
        lo = bv[0]
        hi = bv[1]
        row0 = ghalf * HALF_R
        lo8 = lax.bitwise_and(lo, jnp.int32(-8))
        nch = lax.shift_right_logical(hi - lo8 + (CHUNK - 1), 11)

        def chunk_body(k, carry1):
            base = pl.multiple_of(lo8 + k * CHUNK, 8)
            pltpu.sync_copy(src_hbm.at[pl.ds(base, CHUNK)], ebuf_src)
            pltpu.sync_copy(dst_hbm.at[pl.ds(base, CHUNK)], ebuf_dst)
            nv4 = lax.min(jnp.int32(CHUNK // 64),
                          lax.shift_right_logical(hi - base + 63, 6))

            def vreg_body(i, carry2):
                for u in range(4):
                    off = i * 64 + u * 16
                    sv = ebuf_src[pl.ds(off, 16)]
                    dv = ebuf_dst[pl.ds(off, 16)]
                    e = (base + off) + i16
                    r = dv - row0
                    m = (e >= lo) & (e < hi) & (r >= 0) & (r < HALF_R)
                    plsc.addupdate_scatter(wref, [r, sv], ones_f, mask=m)
                return carry2

            return lax.fori_loop(0, nv4, vreg_body, carry1)

        lax.fori_loop(0, nch, chunk_body, 0)
        return row0

    bufs = (win_a, win_b)
    zero_win(win_a)
    copies = [None, None]
    for q in range(5):
        buf = bufs[q % 2]
        ghalf = c * (N_WIN * 2 // NC) + s + NS * q
        row0 = scatter_half(buf, ghalf)
        copies[q % 2] = pltpu.async_copy(
            buf, a_hbm.at[pl.ds(row0, HALF_R)], sem)
        if q < 4:
            if copies[(q + 1) % 2] is not None:
                copies[(q + 1) % 2].wait()
            zero_win(bufs[(q + 1) % 2])
    copies[1].wait()
    copies[0].wait()


def _build_a(src_pad, dst_pad):
    mesh = plsc.VectorSubcoreMesh(core_axis_name="c", subcore_axis_name="s")
    return pl.kernel(
        _sc_body,
        out_type=jax.ShapeDtypeStruct((N_OUT_PAD, N_IN), jnp.float32),
        mesh=mesh,
        compiler_params=pltpu.CompilerParams(needs_layout_passes=False),
        scratch_types=[
            pltpu.VMEM((CHUNK,), jnp.int32),
            pltpu.VMEM((CHUNK,), jnp.int32),
            pltpu.VMEM((128,), jnp.float32),
            pltpu.VMEM((NS, 128), jnp.float32),
            pltpu.VMEM((128,), jnp.int32),
            pltpu.VMEM((HALF_R, N_IN), jnp.float32),
            pltpu.VMEM((HALF_R, N_IN), jnp.float32),
            pltpu.SemaphoreType.DMA,
            pltpu.VMEM_SHARED((NS, 128), jnp.float32),
        ],
    )(src_pad, dst_pad)


def _tc_body(xt3_ref, a_ref, wf_ref, bf_ref, wl_ref, bl_ref, out_ref, xr_ref):
    g = pl.program_id(0)

    @pl.when(g == 0)
    def _init():
        for ch in range(IN_C):
            xr_ref[:, ch * B:(ch + 1) * B] = xt3_ref[ch].T
        out_ref[...] = jnp.broadcast_to(bl_ref[...], (B, YDIM))

    a_blk = a_ref[...]
    wf = wf_ref[...]
    bf = bf_ref[...]
    m = jnp.dot(a_blk, xr_ref[...], preferred_element_type=jnp.float32)
    acc = jnp.zeros((JB, MIDC, B), jnp.float32)
    for ch in range(IN_C):
        m_c = m[:, ch * B:(ch + 1) * B]
        acc = acc + m_c[:, None, :] * wf[ch][None, :, None]
    h = jnp.tanh(acc + bf[0][None, :, None])
    h2 = h.reshape(JB * MIDC, B)
    wl2 = wl_ref[...].reshape(JB * MIDC, YDIM)
    partial = lax.dot_general(h2, wl2, (((0,), (0,)), ((), ())),
                              preferred_element_type=jnp.float32)
    out_ref[...] += partial


def _dense(xt3, a, wf, bf, wl3, bl):
    return pl.pallas_call(
        _tc_body,
        grid=(N_BLK,),
        in_specs=[
            pl.BlockSpec((IN_C, B, N_IN), lambda g: (0, 0, 0)),
            pl.BlockSpec((JB, N_IN), lambda g: (g, 0)),
            pl.BlockSpec((IN_C, MIDC), lambda g: (0, 0)),
            pl.BlockSpec((1, MIDC), lambda g: (0, 0)),
            pl.BlockSpec((JB, MIDC, YDIM), lambda g: (g, 0, 0)),
            pl.BlockSpec((1, YDIM), lambda g: (0, 0)),
        ],
        out_specs=pl.BlockSpec((B, YDIM), lambda g: (0, 0)),
        out_shape=jax.ShapeDtypeStruct((B, YDIM), jnp.float32),
        scratch_shapes=[pltpu.VMEM((N_IN, IN_C * B), jnp.float32)],
    )(xt3, a, wf, bf, wl3, bl)


@jax.jit
def kernel(x, edge_src, edge_dst, W_fgl, b_fgl, W_lin, b_lin):
    src_pad = jnp.concatenate(
        [edge_src.astype(jnp.int32),
         jnp.zeros((E_ALLOC - E,), jnp.int32)])
    dst_pad = jnp.concatenate(
        [edge_dst.astype(jnp.int32),
         jnp.full((E_ALLOC - E,), N_OUT_PAD - 1, jnp.int32)])
    a = _build_a(src_pad, dst_pad)

    xt3 = jnp.transpose(x, (2, 0, 1))
    wl3 = jnp.pad(W_lin.reshape(N_OUT, MIDC, YDIM),
                  ((0, N_OUT_PAD - N_OUT), (0, 0), (0, 0)))
    return _dense(xt3, a, W_fgl, b_fgl.reshape(1, MIDC), wl3,
                  b_lin.reshape(1, YDIM))


# speedup vs baseline: 118.7705x; 1.0148x over previous
import functools

import jax
import jax.numpy as jnp
from jax import lax
from jax.experimental import pallas as pl
from jax.experimental.pallas import tpu as pltpu
from jax.experimental.pallas import tpu_sc as plsc

B = 128
N_IN = 784
IN_C = 3
MIDC = 16
N_OUT = 10000
E = 160000
YDIM = 10

ROWS_W = 128
N_WIN = 80
N_OUT_PAD = ROWS_W * N_WIN
NC = 2
NS = 16
WPC = N_WIN // NC
HALF_R = 64
CHUNK = 4096
HCH = 2048
EPT = 10240
E_PAD = NS * EPT
E_ALLOC = E_PAD + CHUNK
JB = 1024
N_BLK = N_OUT_PAD // JB


def _sc_body(src_hbm, dst_hbm, a_hbm,
             ebuf_src, ebuf_dst, hist, all_hist, bounds, win_a, win_b, sem,
             shared_hist):
    c = lax.axis_index("c")
    s = lax.axis_index("s")
    i16 = lax.iota(jnp.int32, 16)
    ones_f = jnp.ones((16,), jnp.float32)
    zeros_f = jnp.zeros((16,), jnp.float32)

    for j in range(8):
        hist[pl.ds(j * 16, 16)] = zeros_f

    hbufs = (ebuf_dst, ebuf_src)
    hcopies = [None, None]
    hcopies[0] = pltpu.async_copy(
        dst_hbm.at[pl.ds(pl.multiple_of(s * EPT, 8), HCH)],
        hbufs[0].at[pl.ds(0, HCH)], sem)
    for r in range(EPT // HCH):
        cur = hbufs[r % 2]
        if r + 1 < EPT // HCH:
            base = pl.multiple_of(s * EPT + (r + 1) * HCH, 8)
            hcopies[(r + 1) % 2] = pltpu.async_copy(
                dst_hbm.at[pl.ds(base, HCH)],
                hbufs[(r + 1) % 2].at[pl.ds(0, HCH)], sem)
        hcopies[r % 2].wait()

        def hist_vreg(i, carry1, cur=cur):
            for u in range(4):
                dv = cur[pl.ds(i * 64 + u * 16, 16)]
                bkt = lax.shift_right_logical(dv, 7)
                plsc.addupdate_scatter(hist, [bkt], ones_f)
            return carry1

        lax.fori_loop(0, HCH // 64, hist_vreg, 0)

    pltpu.sync_copy(hist, shared_hist.at[s])
    plsc.subcore_barrier()
    pltpu.sync_copy(shared_hist, all_hist)

    carry = jnp.float32(0)
    for j in range(N_WIN // 16):
        tot = zeros_f
        for t in range(NS):
            tot = tot + all_hist[t, pl.ds(j * 16, 16)]
        cs = lax.cumsum(tot, axis=0)
        bounds[pl.ds(j * 16, 16)] = ((cs - tot) + carry).astype(jnp.int32)
        carry = carry + jnp.sum(tot)
    bounds[pl.ds(N_WIN, 16)] = jnp.broadcast_to(carry.astype(jnp.int32), (16,))

    def zero_win(wref):
        def zrow(i, carry1):
            for j in range(N_IN // 16):
                wref[i, pl.ds(j * 16, 16)] = zeros_f
            return carry1
        lax.fori_loop(0, HALF_R, zrow, 0)

    def scatter_half(wref, ghalf):
        w = lax.shift_right_logical(ghalf, 1)
        bv = bounds[pl.ds(w, 16)]
        lo = bv[0]
        hi = bv[1]
        row0 = ghalf * HALF_R
        lo8 = lax.bitwise_and(lo, jnp.int32(-8))
        nch = lax.shift_right_logical(hi - lo8 + (CHUNK - 1), 12)

        def chunk_body(k, carry1):
            base = pl.multiple_of(lo8 + k * CHUNK, 8)
            pltpu.sync_copy(src_hbm.at[pl.ds(base, CHUNK)], ebuf_src)
            pltpu.sync_copy(dst_hbm.at[pl.ds(base, CHUNK)], ebuf_dst)
            nv4 = lax.min(jnp.int32(CHUNK // 64),
                          lax.shift_right_logical(hi - base + 63, 6))

            def vreg_body(i, carry2):
                for u in range(4):
                    off = i * 64 + u * 16
                    sv = ebuf_src[pl.ds(off, 16)]
                    dv = ebuf_dst[pl.ds(off, 16)]
                    e = (base + off) + i16
                    r = dv - row0
                    m = (e >= lo) & (e < hi) & (r >= 0) & (r < HALF_R)
                    plsc.addupdate_scatter(wref, [r, sv], ones_f, mask=m)
                return carry2

            return lax.fori_loop(0, nv4, vreg_body, carry1)

        lax.fori_loop(0, nch, chunk_body, 0)
        return row0

    bufs = (win_a, win_b)
    zero_win(win_a)
    copies = [None, None]
    for q in range(5):
        buf = bufs[q % 2]
        ghalf = c * (N_WIN * 2 // NC) + s + NS * q
        row0 = scatter_half(buf, ghalf)
        copies[q % 2] = pltpu.async_copy(
            buf, a_hbm.at[pl.ds(row0, HALF_R)], sem)
        if q < 4:
            if copies[(q + 1) % 2] is not None:
                copies[(q + 1) % 2].wait()
            zero_win(bufs[(q + 1) % 2])
    copies[1].wait()
    copies[0].wait()


def _build_a(src_pad, dst_pad):
    mesh = plsc.VectorSubcoreMesh(core_axis_name="c", subcore_axis_name="s")
    return pl.kernel(
        _sc_body,
        out_type=jax.ShapeDtypeStruct((N_OUT_PAD, N_IN), jnp.float32),
        mesh=mesh,
        compiler_params=pltpu.CompilerParams(needs_layout_passes=False),
        scratch_types=[
            pltpu.VMEM((CHUNK,), jnp.int32),
            pltpu.VMEM((CHUNK,), jnp.int32),
            pltpu.VMEM((128,), jnp.float32),
            pltpu.VMEM((NS, 128), jnp.float32),
            pltpu.VMEM((128,), jnp.int32),
            pltpu.VMEM((HALF_R, N_IN), jnp.float32),
            pltpu.VMEM((HALF_R, N_IN), jnp.float32),
            pltpu.SemaphoreType.DMA,
            pltpu.VMEM_SHARED((NS, 128), jnp.float32),
        ],
    )(src_pad, dst_pad)


def _tc_body(xt3_ref, a_ref, wf_ref, bf_ref, wl_ref, bl_ref, out_ref, xr_ref):
    g = pl.program_id(0)

    @pl.when(g == 0)
    def _init():
        for ch in range(IN_C):
            xr_ref[:, ch * B:(ch + 1) * B] = xt3_ref[ch].T
        out_ref[...] = jnp.broadcast_to(bl_ref[...], (B, YDIM))

    a_blk = a_ref[...]
    wf = wf_ref[...]
    bf = bf_ref[...]
    m = jnp.dot(a_blk, xr_ref[...], preferred_element_type=jnp.float32)
    acc = jnp.zeros((JB, MIDC, B), jnp.float32)
    for ch in range(IN_C):
        m_c = m[:, ch * B:(ch + 1) * B]
        acc = acc + m_c[:, None, :] * wf[ch][None, :, None]
    h = jnp.tanh(acc + bf[0][None, :, None])
    h2 = h.reshape(JB * MIDC, B)
    wl2 = wl_ref[...].reshape(JB * MIDC, YDIM)
    partial = lax.dot_general(h2, wl2, (((0,), (0,)), ((), ())),
                              preferred_element_type=jnp.float32)
    out_ref[...] += partial


def _dense(xt3, a, wf, bf, wl3, bl):
    return pl.pallas_call(
        _tc_body,
        grid=(N_BLK,),
        in_specs=[
            pl.BlockSpec((IN_C, B, N_IN), lambda g: (0, 0, 0)),
            pl.BlockSpec((JB, N_IN), lambda g: (g, 0)),
            pl.BlockSpec((IN_C, MIDC), lambda g: (0, 0)),
            pl.BlockSpec((1, MIDC), lambda g: (0, 0)),
            pl.BlockSpec((JB, MIDC, YDIM), lambda g: (g, 0, 0)),
            pl.BlockSpec((1, YDIM), lambda g: (0, 0)),
        ],
        out_specs=pl.BlockSpec((B, YDIM), lambda g: (0, 0)),
        out_shape=jax.ShapeDtypeStruct((B, YDIM), jnp.float32),
        scratch_shapes=[pltpu.VMEM((N_IN, IN_C * B), jnp.float32)],
    )(xt3, a, wf, bf, wl3, bl)


@jax.jit
def kernel(x, edge_src, edge_dst, W_fgl, b_fgl, W_lin, b_lin):
    src_pad = jnp.concatenate(
        [edge_src.astype(jnp.int32),
         jnp.zeros((E_ALLOC - E,), jnp.int32)])
    dst_pad = jnp.concatenate(
        [edge_dst.astype(jnp.int32),
         jnp.full((E_ALLOC - E,), N_OUT_PAD - 1, jnp.int32)])
    a = _build_a(src_pad, dst_pad)

    xt3 = jnp.transpose(x, (2, 0, 1))
    wl3 = jnp.pad(W_lin.reshape(N_OUT, MIDC, YDIM),
                  ((0, N_OUT_PAD - N_OUT), (0, 0), (0, 0)))
    return _dense(xt3, a, W_fgl, b_fgl.reshape(1, MIDC), wl3,
                  b_lin.reshape(1, YDIM))
